# baseline (speedup 1.0000x reference)
import jax
import jax.numpy as jnp
from jax import lax
from jax.experimental import pallas as pl
from jax.experimental.pallas import tpu as pltpu
from jax.experimental.pallas import tpu_sc as plsc

_B = 128
_V = 100000
_L = 16
_RB = 8
_NSPLIT = 4
_SC_ROWS = 32
_TC_ROWS = _B - _SC_ROWS
_VREGS = _V // _L
_UNROLL = 5
_ITERS = _VREGS // _UNROLL


def _count_body(tgt_ref, *refs):
    b = pl.program_id(0)
    x_refs, out_ref = refs[:_NSPLIT], refs[_NSPLIT]
    tgt = tgt_ref[...]
    col = lax.broadcasted_iota(jnp.int32, (_RB, _V), 1)
    part = None
    for p in range(_NSPLIT):
        x = x_refs[p][...]
        tgtp = tgt[p * _RB:(p + 1) * _RB]
        hit = col == tgtp
        t = jnp.sum(jnp.where(hit, x, 0.0), axis=1, keepdims=True)
        pred = (x > t) | ((x == t) & (col < tgtp))
        rank = jnp.sum(pred.astype(jnp.float32), axis=1, keepdims=True)
        apk = ((rank == 0.0).astype(jnp.float32)
               + (rank == 1.0).astype(jnp.float32) * 0.5
               + (rank == 2.0).astype(jnp.float32) * (1.0 / 3.0))
        s = jnp.sum(apk, axis=(0, 1), keepdims=True)
        part = s if part is None else part + s

    @pl.when(b == 0)
    def _():
        out_ref[...] = part

    @pl.when(b != 0)
    def _():
        out_ref[...] += part


def _tc_count(x, target):
    rows_per_step = _RB * _NSPLIT
    grid = _TC_ROWS // rows_per_step
    x_specs = [
        pl.BlockSpec((_RB, _V), lambda b, p=p: (_NSPLIT * b + p, 0))
        for p in range(_NSPLIT)
    ]
    return pl.pallas_call(
        _count_body,
        grid=(grid,),
        in_specs=[
            pl.BlockSpec((rows_per_step, 1), lambda b: (b, 0)),
            *x_specs,
        ],
        out_specs=pl.BlockSpec((1, 1), lambda b: (0, 0)),
        out_shape=jax.ShapeDtypeStruct((1, 1), jnp.float32),
        compiler_params=pltpu.CompilerParams(
            dimension_semantics=("arbitrary",),
            vmem_limit_bytes=100 * 1024 * 1024,
        ),
    )(target, *([x] * _NSPLIT))


def _sc_count(x, target):
    nc = plsc.get_sparse_core_info().num_cores

    def body(x_hbm, tgt_hbm, out_hbm, row_v, tgt_v, apk_v):
        sid = lax.axis_index("s")
        cid = lax.axis_index("c")
        wid = sid * nc + cid
        row = _TC_ROWS + wid
        lane = jnp.remainder(wid, 8)

        al = _TC_ROWS + (wid // 8) * 8
        pltpu.sync_copy(tgt_hbm.at[pl.ds(al, 8)], tgt_v.at[pl.ds(0, 8)])
        idx = jnp.broadcast_to(lane, (_L,)).astype(jnp.int32)
        tgt_vec = plsc.load_gather(tgt_v, [idx])

        pltpu.sync_copy(x_hbm.at[row, :], row_v)

        iota = lax.iota(jnp.int32, _L)

        def p1(i, acc):
            for u in range(_UNROLL):
                o = i * (_UNROLL * _L) + u * _L
                v = row_v[pl.ds(o, _L)]
                colv = o + iota
                acc = acc + jnp.where(colv == tgt_vec, v, 0.0)
            return acc

        acc_t = lax.fori_loop(0, _ITERS, p1, jnp.zeros((_L,), jnp.float32))
        t = jnp.sum(acc_t, axis=0)
        t_vec = jnp.broadcast_to(t, (_L,))

        one = jnp.ones((_L,), jnp.float32)
        zero = jnp.zeros((_L,), jnp.float32)

        def p2(i, acc):
            for u in range(_UNROLL):
                o = i * (_UNROLL * _L) + u * _L
                v = row_v[pl.ds(o, _L)]
                colv = o + iota
                pred = (v > t_vec) | ((v == t_vec) & (colv < tgt_vec))
                acc = acc + jnp.where(pred, one, zero)
            return acc

        acc_c = lax.fori_loop(0, _ITERS, p2, jnp.zeros((_L,), jnp.float32))
        rank = jnp.sum(acc_c, axis=0)
        apk = (jnp.where(rank == 0.0, 1.0, 0.0)
               + jnp.where(rank == 1.0, 0.5, 0.0)
               + jnp.where(rank == 2.0, 1.0 / 3.0, 0.0))
        lane0 = jnp.where(iota == 0, 1.0, 0.0)
        apk_v[...] = jnp.broadcast_to(apk, (_L,)) * lane0
        pltpu.sync_copy(apk_v, out_hbm.at[wid])

    mesh = plsc.VectorSubcoreMesh(core_axis_name="c", subcore_axis_name="s")
    k = pl.kernel(
        body,
        mesh=mesh,
        out_type=jax.ShapeDtypeStruct((_SC_ROWS, _L), jnp.float32),
        scratch_types=[
            pltpu.VMEM((_V,), jnp.float32),
            pltpu.VMEM((_L,), jnp.int32),
            pltpu.VMEM((_L,), jnp.float32),
        ],
        compiler_params=pltpu.CompilerParams(
            use_tc_tiling_on_sc=True, needs_layout_passes=False),
    )
    return k(x, target)


def kernel(input, target):
    tc = _tc_count(input, target.reshape(_B, 1))
    sc = _sc_count(input, target)
    return (tc[0, 0] + jnp.sum(sc[:, 0])) * (1.0 / _B)


# speedup vs baseline: 1.0732x; 1.0732x over previous
import functools

import jax
import jax.numpy as jnp
from jax import lax
from jax.experimental import pallas as pl
from jax.experimental.pallas import tpu as pltpu
from jax.experimental.pallas import tpu_sc as plsc

_B = 128
_V = 100000
_L = 16
_RB = 8
_ROWS_PER_SUBCORE = 16
_ACTIVE_SUBCORES = _B // _ROWS_PER_SUBCORE

def _sc_gather(tab, target):
    nc = plsc.get_sparse_core_info().num_cores

    def body(tab_hbm, tgt_hbm, t_hbm, tgt_v, pad_v, t_v, sem):
        wid = lax.axis_index("s") * nc + lax.axis_index("c")

        @pl.when(wid < _ACTIVE_SUBCORES)
        def _():
            base = wid * _ROWS_PER_SUBCORE
            pltpu.sync_copy(tgt_hbm.at[pl.ds(base, _ROWS_PER_SUBCORE)], tgt_v)
            tv = tgt_v[...]
            copies = []
            for r in range(_ROWS_PER_SUBCORE):
                c_al = (tv[r] // 8) * 8
                copies.append(pltpu.async_copy(
                    tab_hbm.at[base + r, pl.ds(c_al, 8)],
                    pad_v.at[pl.ds(r * 8, 8)], sem))
            for c in copies:
                c.wait()
            idx = lax.iota(jnp.int32, _L) * 8 + lax.bitwise_and(tgt_v[...], 7)
            t_v[...] = plsc.load_gather(pad_v, [idx])
            pltpu.sync_copy(t_v, t_hbm.at[pl.ds(base, _ROWS_PER_SUBCORE)])

    mesh = plsc.VectorSubcoreMesh(core_axis_name="c", subcore_axis_name="s")
    k = pl.kernel(
        body,
        mesh=mesh,
        out_type=jax.ShapeDtypeStruct((_B,), jnp.float32),
        scratch_types=[
            pltpu.VMEM((_L,), jnp.int32),
            pltpu.VMEM((_L * 8,), jnp.float32),
            pltpu.VMEM((_L,), jnp.float32),
            pltpu.SemaphoreType.DMA,
        ],
        compiler_params=pltpu.CompilerParams(
            use_tc_tiling_on_sc=True, needs_layout_passes=False),
    )
    return k(tab, target)


_NSPLIT = 4


def _count_body(tgt_ref, t_ref, *refs):
    b = pl.program_id(0)
    x_refs, out_ref = refs[:_NSPLIT], refs[_NSPLIT]
    t = t_ref[...]
    tgt = tgt_ref[...]
    col = lax.broadcasted_iota(jnp.int32, (_RB, _V), 1)
    part = None
    for p in range(_NSPLIT):
        x = x_refs[p][...]
        tp = t[p * _RB:(p + 1) * _RB]
        tgtp = tgt[p * _RB:(p + 1) * _RB]
        pred = (x > tp) | ((x == tp) & (col < tgtp))
        rank = jnp.sum(pred.astype(jnp.float32), axis=1, keepdims=True)
        apk = ((rank == 0.0).astype(jnp.float32)
               + (rank == 1.0).astype(jnp.float32) * 0.5
               + (rank == 2.0).astype(jnp.float32) * (1.0 / 3.0))
        s = jnp.sum(apk, axis=(0, 1), keepdims=True)
        part = s if part is None else part + s
    part = part * (1.0 / _B)

    @pl.when(b == 0)
    def _():
        out_ref[...] = part

    @pl.when(b != 0)
    def _():
        out_ref[...] += part


def _tc_count(x, t, target):
    rows_per_step = _RB * _NSPLIT
    grid = _B // rows_per_step
    x_specs = [
        pl.BlockSpec((_RB, _V), lambda b, p=p: (_NSPLIT * b + p, 0))
        for p in range(_NSPLIT)
    ]
    return pl.pallas_call(
        _count_body,
        grid=(grid,),
        in_specs=[
            pl.BlockSpec((rows_per_step, 1), lambda b: (b, 0)),
            pl.BlockSpec((rows_per_step, 1), lambda b: (b, 0)),
            *x_specs,
        ],
        out_specs=pl.BlockSpec((1, 1), lambda b: (0, 0)),
        out_shape=jax.ShapeDtypeStruct((1, 1), jnp.float32),
        compiler_params=pltpu.CompilerParams(
            dimension_semantics=("arbitrary",),
            vmem_limit_bytes=100 * 1024 * 1024,
        ),
    )(target, t, *([x] * _NSPLIT))


def kernel(input, target):
    t = _sc_gather(input, target)
    res = _tc_count(input, t.reshape(_B, 1), target.reshape(_B, 1))
    return res[0, 0]
